# initial kernel scaffold (unmeasured)
import jax
import jax.numpy as jnp
from jax import lax
from jax.experimental import pallas as pl
from jax.experimental.pallas import tpu as pltpu

N_DEV = 16


def kernel(x, w_mat):
    M, K = x.shape
    _, N = w_mat.shape
    CH = M // N_DEV

    def body(x_ref, w_ref, out_ref, acc_ref, rs_buf, ag_buf,
             rs_send, rs_recv, ag_send, ag_recv,
             rs_credit, ag_credit, store_sem):
        d = lax.axis_index("i")
        left = (d + N_DEV - 1) % N_DEV
        right = (d + 1) % N_DEV

        barrier = pltpu.get_barrier_semaphore()
        for nbr in (left, right):
            pl.semaphore_signal(barrier, inc=1, device_id=(nbr,),
                                device_id_type=pl.DeviceIdType.MESH)
        pl.semaphore_wait(barrier, 2)

        pl.semaphore_signal(rs_credit, inc=1, device_id=(left,),
                            device_id_type=pl.DeviceIdType.MESH)
        pl.semaphore_signal(ag_credit, inc=1, device_id=(left,),
                            device_id_type=pl.DeviceIdType.MESH)

        def partial_chunk(c):
            return jnp.dot(x_ref[pl.ds(c * CH, CH), :], w_ref[:, :],
                           preferred_element_type=jnp.float32)

        rs_buf[1, :, :] = partial_chunk(d)
        for s in range(N_DEV - 1):
            send_slot = (s + 1) % 2
            recv_slot = s % 2
            c_recv = (d - s - 1) % N_DEV
            pl.semaphore_wait(rs_credit, 1)
            rdma = pltpu.make_async_remote_copy(
                src_ref=rs_buf.at[send_slot],
                dst_ref=rs_buf.at[recv_slot],
                send_sem=rs_send.at[send_slot],
                recv_sem=rs_recv.at[recv_slot],
                device_id=(right,),
                device_id_type=pl.DeviceIdType.MESH,
            )
            rdma.start()
            acc_ref[:, :] = partial_chunk(c_recv)
            rdma.wait_recv()
            rs_buf[recv_slot, :, :] = rs_buf[recv_slot, :, :] + acc_ref[:, :]
            rdma.wait_send()
            if s < N_DEV - 2:
                pl.semaphore_signal(rs_credit, inc=1, device_id=(left,),
                                    device_id_type=pl.DeviceIdType.MESH)

        own = (d + 1) % N_DEV
        v = rs_buf[0, :, :]
        ag_buf[0, :, :] = v * (1.0 / (1.0 + jnp.exp(-v)))
        cp = pltpu.make_async_copy(
            ag_buf.at[0], out_ref.at[pl.ds(own * CH, CH), :], store_sem)
        cp.start()
        cp.wait()

        for h in range(N_DEV - 1):
            send_slot = h % 2
            recv_slot = (h + 1) % 2
            c = (d - h) % N_DEV
            pl.semaphore_wait(ag_credit, 1)
            rdma = pltpu.make_async_remote_copy(
                src_ref=ag_buf.at[send_slot],
                dst_ref=ag_buf.at[recv_slot],
                send_sem=ag_send.at[send_slot],
                recv_sem=ag_recv.at[recv_slot],
                device_id=(right,),
                device_id_type=pl.DeviceIdType.MESH,
            )
            rdma.start()
            rdma.wait_recv()
            cp = pltpu.make_async_copy(
                ag_buf.at[recv_slot], out_ref.at[pl.ds(c * CH, CH), :],
                store_sem)
            cp.start()
            cp.wait()
            rdma.wait_send()
            if h < N_DEV - 2:
                pl.semaphore_signal(ag_credit, inc=1, device_id=(left,),
                                    device_id_type=pl.DeviceIdType.MESH)

    return pl.pallas_call(
        body,
        out_shape=jax.ShapeDtypeStruct((M, N), jnp.float32),
        in_specs=[
            pl.BlockSpec(memory_space=pltpu.VMEM),
            pl.BlockSpec(memory_space=pltpu.VMEM),
        ],
        out_specs=pl.BlockSpec(memory_space=pltpu.ANY),
        scratch_shapes=[
            pltpu.VMEM((CH, N), jnp.float32),
            pltpu.VMEM((2, CH, N), jnp.float32),
            pltpu.VMEM((2, CH, N), jnp.float32),
            pltpu.SemaphoreType.DMA((2,)),
            pltpu.SemaphoreType.DMA((2,)),
            pltpu.SemaphoreType.DMA((2,)),
            pltpu.SemaphoreType.DMA((2,)),
            pltpu.SemaphoreType.REGULAR,
            pltpu.SemaphoreType.REGULAR,
            pltpu.SemaphoreType.DMA,
        ],
        compiler_params=pltpu.CompilerParams(collective_id=0),
    )(x, w_mat)


# baseline (device time: 3013756 ns/iter reference)
import jax
import jax.numpy as jnp
from jax import lax
from jax.experimental import pallas as pl
from jax.experimental.pallas import tpu as pltpu

N_DEV = 16


def kernel(x, w_mat):
    M, K = x.shape
    _, N = w_mat.shape
    CH = M // N_DEV

    def body(x_ref, w_ref, out_ref, acc_ref, rs_buf, ag_buf,
             rs_send, rs_recv, ag_send, ag_recv,
             rs_credit, ag_credit, store_sem):
        d = lax.axis_index("i")
        left = (d + N_DEV - 1) % N_DEV
        right = (d + 1) % N_DEV

        barrier = pltpu.get_barrier_semaphore()
        for nbr in (left, right):
            pl.semaphore_signal(barrier, inc=1, device_id=(nbr,),
                                device_id_type=pl.DeviceIdType.MESH)
        pl.semaphore_wait(barrier, 2)

        pl.semaphore_signal(rs_credit, inc=1, device_id=(left,),
                            device_id_type=pl.DeviceIdType.MESH)
        pl.semaphore_signal(ag_credit, inc=1, device_id=(left,),
                            device_id_type=pl.DeviceIdType.MESH)

        def partial_chunk(c):
            return jnp.dot(x_ref[pl.ds(c * CH, CH), :], w_ref[:, :],
                           preferred_element_type=jnp.float32)

        rs_buf[1, :, :] = partial_chunk(d)
        for s in range(N_DEV - 1):
            send_slot = (s + 1) % 2
            recv_slot = s % 2
            c_recv = (d - s - 1) % N_DEV
            pl.semaphore_wait(rs_credit, 1)
            rdma = pltpu.make_async_remote_copy(
                src_ref=rs_buf.at[send_slot],
                dst_ref=rs_buf.at[recv_slot],
                send_sem=rs_send.at[send_slot],
                recv_sem=rs_recv.at[recv_slot],
                device_id=(right,),
                device_id_type=pl.DeviceIdType.MESH,
            )
            rdma.start()
            acc_ref[:, :] = partial_chunk(c_recv)
            rdma.wait_recv()
            rs_buf[recv_slot, :, :] = rs_buf[recv_slot, :, :] + acc_ref[:, :]
            rdma.wait_send()
            if s < N_DEV - 2:
                pl.semaphore_signal(rs_credit, inc=1, device_id=(left,),
                                    device_id_type=pl.DeviceIdType.MESH)

        own = (d + 1) % N_DEV
        v = rs_buf[0, :, :]
        ag_buf[0, :, :] = v * (1.0 / (1.0 + jnp.exp(-v)))
        cp = pltpu.make_async_copy(
            ag_buf.at[0], out_ref.at[pl.ds(own * CH, CH), :], store_sem)
        cp.start()
        cp.wait()

        for h in range(N_DEV - 1):
            send_slot = h % 2
            recv_slot = (h + 1) % 2
            c = (d - h) % N_DEV
            pl.semaphore_wait(ag_credit, 1)
            rdma = pltpu.make_async_remote_copy(
                src_ref=ag_buf.at[send_slot],
                dst_ref=ag_buf.at[recv_slot],
                send_sem=ag_send.at[send_slot],
                recv_sem=ag_recv.at[recv_slot],
                device_id=(right,),
                device_id_type=pl.DeviceIdType.MESH,
            )
            rdma.start()
            rdma.wait_recv()
            cp = pltpu.make_async_copy(
                ag_buf.at[recv_slot], out_ref.at[pl.ds(c * CH, CH), :],
                store_sem)
            cp.start()
            cp.wait()
            rdma.wait_send()
            if h < N_DEV - 2:
                pl.semaphore_signal(ag_credit, inc=1, device_id=(left,),
                                    device_id_type=pl.DeviceIdType.MESH)

    return pl.pallas_call(
        body,
        out_shape=jax.ShapeDtypeStruct((M, N), jnp.float32),
        in_specs=[
            pl.BlockSpec(memory_space=pltpu.VMEM),
            pl.BlockSpec(memory_space=pltpu.VMEM),
        ],
        out_specs=pl.BlockSpec(memory_space=pl.ANY),
        scratch_shapes=[
            pltpu.VMEM((CH, N), jnp.float32),
            pltpu.VMEM((2, CH, N), jnp.float32),
            pltpu.VMEM((2, CH, N), jnp.float32),
            pltpu.SemaphoreType.DMA((2,)),
            pltpu.SemaphoreType.DMA((2,)),
            pltpu.SemaphoreType.DMA((2,)),
            pltpu.SemaphoreType.DMA((2,)),
            pltpu.SemaphoreType.REGULAR,
            pltpu.SemaphoreType.REGULAR,
            pltpu.SemaphoreType.DMA,
        ],
        compiler_params=pltpu.CompilerParams(
            collective_id=0,
            vmem_limit_bytes=100 * 1024 * 1024,
        ),
    )(x, w_mat)


# device time: 1661771 ns/iter; 1.8136x vs baseline; 1.8136x over previous
import jax
import jax.numpy as jnp
from jax import lax
from jax.experimental import pallas as pl
from jax.experimental.pallas import tpu as pltpu

N_DEV = 16


def kernel(x, w_mat):
    M, K = x.shape
    _, N = w_mat.shape
    CH = M // N_DEV
    H = N // 2

    def body(x_ref, w_ref, out_ref,
             acc_f, acc_b, rs_f, rs_b, ag_f, ag_b,
             rs_send_f, rs_recv_f, rs_send_b, rs_recv_b,
             ag_send_f, ag_recv_f, ag_send_b, ag_recv_b,
             rs_credit_f, rs_credit_b, ag_credit_f, ag_credit_b,
             store_f, store_b):
        d = lax.axis_index("i")
        left = (d + N_DEV - 1) % N_DEV
        right = (d + 1) % N_DEV

        barrier = pltpu.get_barrier_semaphore()
        for nbr in (left, right):
            pl.semaphore_signal(barrier, inc=1, device_id=(nbr,),
                                device_id_type=pl.DeviceIdType.MESH)
        pl.semaphore_wait(barrier, 2)

        def grant(sem, to):
            pl.semaphore_signal(sem, inc=1, device_id=(to,),
                                device_id_type=pl.DeviceIdType.MESH)

        grant(rs_credit_f, left)
        grant(rs_credit_b, right)
        grant(ag_credit_f, left)
        grant(ag_credit_b, right)

        def pchunk(c, lo):
            return jnp.dot(x_ref[pl.ds(c * CH, CH), :], w_ref[:, lo:lo + H],
                           preferred_element_type=jnp.float32)

        def rcopy(buf, send_sems, recv_sems, send_slot, recv_slot, to):
            return pltpu.make_async_remote_copy(
                src_ref=buf.at[send_slot],
                dst_ref=buf.at[recv_slot],
                send_sem=send_sems.at[send_slot],
                recv_sem=recv_sems.at[recv_slot],
                device_id=(to,),
                device_id_type=pl.DeviceIdType.MESH,
            )

        rs_f[1, :, :] = pchunk(d, 0)
        rs_b[1, :, :] = pchunk(d, H)
        for s in range(N_DEV - 1):
            send_slot = (s + 1) % 2
            recv_slot = s % 2
            cf = (d - s - 1) % N_DEV
            cb = (d + s + 1) % N_DEV
            pl.semaphore_wait(rs_credit_f, 1)
            rf = rcopy(rs_f, rs_send_f, rs_recv_f, send_slot, recv_slot, right)
            rf.start()
            pl.semaphore_wait(rs_credit_b, 1)
            rb = rcopy(rs_b, rs_send_b, rs_recv_b, send_slot, recv_slot, left)
            rb.start()
            acc_f[:, :] = pchunk(cf, 0)
            acc_b[:, :] = pchunk(cb, H)
            rf.wait_recv()
            rs_f[recv_slot, :, :] = rs_f[recv_slot, :, :] + acc_f[:, :]
            rb.wait_recv()
            rs_b[recv_slot, :, :] = rs_b[recv_slot, :, :] + acc_b[:, :]
            rf.wait_send()
            rb.wait_send()
            if s < N_DEV - 2:
                grant(rs_credit_f, left)
                grant(rs_credit_b, right)

        own_f = (d + 1) % N_DEV
        own_b = (d + N_DEV - 1) % N_DEV
        vf = rs_f[0, :, :]
        ag_f[0, :, :] = vf * (1.0 / (1.0 + jnp.exp(-vf)))
        vb = rs_b[0, :, :]
        ag_b[0, :, :] = vb * (1.0 / (1.0 + jnp.exp(-vb)))
        cpf = pltpu.make_async_copy(
            ag_f.at[0], out_ref.at[pl.ds(own_f * CH, CH), pl.ds(0, H)],
            store_f)
        cpb = pltpu.make_async_copy(
            ag_b.at[0], out_ref.at[pl.ds(own_b * CH, CH), pl.ds(H, H)],
            store_b)
        cpf.start()
        cpb.start()
        cpf.wait()
        cpb.wait()

        for h in range(N_DEV - 1):
            send_slot = h % 2
            recv_slot = (h + 1) % 2
            cf = (d - h) % N_DEV
            cb = (d + h) % N_DEV
            pl.semaphore_wait(ag_credit_f, 1)
            rf = rcopy(ag_f, ag_send_f, ag_recv_f, send_slot, recv_slot, right)
            rf.start()
            pl.semaphore_wait(ag_credit_b, 1)
            rb = rcopy(ag_b, ag_send_b, ag_recv_b, send_slot, recv_slot, left)
            rb.start()
            rf.wait_recv()
            cpf = pltpu.make_async_copy(
                ag_f.at[recv_slot],
                out_ref.at[pl.ds(cf * CH, CH), pl.ds(0, H)], store_f)
            cpf.start()
            rb.wait_recv()
            cpb = pltpu.make_async_copy(
                ag_b.at[recv_slot],
                out_ref.at[pl.ds(cb * CH, CH), pl.ds(H, H)], store_b)
            cpb.start()
            cpf.wait()
            cpb.wait()
            rf.wait_send()
            rb.wait_send()
            if h < N_DEV - 2:
                grant(ag_credit_f, left)
                grant(ag_credit_b, right)

    return pl.pallas_call(
        body,
        out_shape=jax.ShapeDtypeStruct((M, N), jnp.float32),
        in_specs=[
            pl.BlockSpec(memory_space=pltpu.VMEM),
            pl.BlockSpec(memory_space=pltpu.VMEM),
        ],
        out_specs=pl.BlockSpec(memory_space=pl.ANY),
        scratch_shapes=[
            pltpu.VMEM((CH, H), jnp.float32),
            pltpu.VMEM((CH, H), jnp.float32),
            pltpu.VMEM((2, CH, H), jnp.float32),
            pltpu.VMEM((2, CH, H), jnp.float32),
            pltpu.VMEM((2, CH, H), jnp.float32),
            pltpu.VMEM((2, CH, H), jnp.float32),
            pltpu.SemaphoreType.DMA((2,)),
            pltpu.SemaphoreType.DMA((2,)),
            pltpu.SemaphoreType.DMA((2,)),
            pltpu.SemaphoreType.DMA((2,)),
            pltpu.SemaphoreType.DMA((2,)),
            pltpu.SemaphoreType.DMA((2,)),
            pltpu.SemaphoreType.DMA((2,)),
            pltpu.SemaphoreType.DMA((2,)),
            pltpu.SemaphoreType.REGULAR,
            pltpu.SemaphoreType.REGULAR,
            pltpu.SemaphoreType.REGULAR,
            pltpu.SemaphoreType.REGULAR,
            pltpu.SemaphoreType.DMA,
            pltpu.SemaphoreType.DMA,
        ],
        compiler_params=pltpu.CompilerParams(
            collective_id=0,
            vmem_limit_bytes=100 * 1024 * 1024,
        ),
    )(x, w_mat)


# device time: 1617902 ns/iter; 1.8628x vs baseline; 1.0271x over previous
import jax
import jax.numpy as jnp
from jax import lax
from jax.experimental import pallas as pl
from jax.experimental.pallas import tpu as pltpu

N_DEV = 16


def kernel(x, w_mat):
    M, K = x.shape
    _, N = w_mat.shape
    CH = M // N_DEV
    H = N // 2

    def body(x_ref, w_ref, out_ref,
             acc_f, acc_b, rs_f, rs_b, ag_f, ag_b,
             rs_send_f, rs_recv_f, rs_send_b, rs_recv_b,
             ag_send_f, ag_recv_f, ag_send_b, ag_recv_b,
             rs_credit_f, rs_credit_b, ag_credit_f, ag_credit_b,
             store_f, store_b):
        d = lax.axis_index("i")
        left = (d + N_DEV - 1) % N_DEV
        right = (d + 1) % N_DEV

        barrier = pltpu.get_barrier_semaphore()
        for nbr in (left, right):
            pl.semaphore_signal(barrier, inc=1, device_id=(nbr,),
                                device_id_type=pl.DeviceIdType.MESH)
        pl.semaphore_wait(barrier, 2)

        def grant(sem, to):
            pl.semaphore_signal(sem, inc=1, device_id=(to,),
                                device_id_type=pl.DeviceIdType.MESH)

        grant(rs_credit_f, left)
        grant(rs_credit_b, right)
        grant(ag_credit_f, left)
        grant(ag_credit_b, right)

        def pchunk(c, lo):
            return jnp.dot(x_ref[pl.ds(c * CH, CH), :], w_ref[:, lo:lo + H],
                           preferred_element_type=jnp.float32)

        def rcopy(buf, send_sems, recv_sems, send_slot, recv_slot, to):
            return pltpu.make_async_remote_copy(
                src_ref=buf.at[send_slot],
                dst_ref=buf.at[recv_slot],
                send_sem=send_sems.at[send_slot],
                recv_sem=recv_sems.at[recv_slot],
                device_id=(to,),
                device_id_type=pl.DeviceIdType.MESH,
            )

        rs_f[1, :, :] = pchunk(d, 0)
        rs_b[1, :, :] = pchunk(d, H)
        for s in range(N_DEV - 1):
            send_slot = (s + 1) % 2
            recv_slot = s % 2
            cf = (d - s - 1) % N_DEV
            cb = (d + s + 1) % N_DEV
            pl.semaphore_wait(rs_credit_f, 1)
            rf = rcopy(rs_f, rs_send_f, rs_recv_f, send_slot, recv_slot, right)
            rf.start()
            pl.semaphore_wait(rs_credit_b, 1)
            rb = rcopy(rs_b, rs_send_b, rs_recv_b, send_slot, recv_slot, left)
            rb.start()
            acc_f[:, :] = pchunk(cf, 0)
            acc_b[:, :] = pchunk(cb, H)
            if s % 2 == 0:
                rf.wait_recv()
                rs_f[recv_slot, :, :] = rs_f[recv_slot, :, :] + acc_f[:, :]
                rb.wait_recv()
                rs_b[recv_slot, :, :] = rs_b[recv_slot, :, :] + acc_b[:, :]
            else:
                rb.wait_recv()
                rs_b[recv_slot, :, :] = rs_b[recv_slot, :, :] + acc_b[:, :]
                rf.wait_recv()
                rs_f[recv_slot, :, :] = rs_f[recv_slot, :, :] + acc_f[:, :]
            rf.wait_send()
            rb.wait_send()
            if s < N_DEV - 2:
                grant(rs_credit_f, left)
                grant(rs_credit_b, right)

        own_f = (d + 1) % N_DEV
        own_b = (d + N_DEV - 1) % N_DEV
        vf = rs_f[0, :, :]
        ag_f[0, :, :] = vf * (1.0 / (1.0 + jnp.exp(-vf)))
        vb = rs_b[0, :, :]
        ag_b[0, :, :] = vb * (1.0 / (1.0 + jnp.exp(-vb)))
        cpf_prev = pltpu.make_async_copy(
            ag_f.at[0], out_ref.at[pl.ds(own_f * CH, CH), pl.ds(0, H)],
            store_f.at[1])
        cpb_prev = pltpu.make_async_copy(
            ag_b.at[0], out_ref.at[pl.ds(own_b * CH, CH), pl.ds(H, H)],
            store_b.at[1])
        cpf_prev.start()
        cpb_prev.start()

        for h in range(N_DEV - 1):
            send_slot = h % 2
            recv_slot = (h + 1) % 2
            cf = (d - h) % N_DEV
            cb = (d + h) % N_DEV
            pl.semaphore_wait(ag_credit_f, 1)
            rf = rcopy(ag_f, ag_send_f, ag_recv_f, send_slot, recv_slot, right)
            rf.start()
            pl.semaphore_wait(ag_credit_b, 1)
            rb = rcopy(ag_b, ag_send_b, ag_recv_b, send_slot, recv_slot, left)
            rb.start()
            rf.wait_recv()
            cpf = pltpu.make_async_copy(
                ag_f.at[recv_slot],
                out_ref.at[pl.ds(cf * CH, CH), pl.ds(0, H)],
                store_f.at[h % 2])
            cpf.start()
            rb.wait_recv()
            cpb = pltpu.make_async_copy(
                ag_b.at[recv_slot],
                out_ref.at[pl.ds(cb * CH, CH), pl.ds(H, H)],
                store_b.at[h % 2])
            cpb.start()
            cpf_prev.wait()
            cpb_prev.wait()
            cpf_prev, cpb_prev = cpf, cpb
            rf.wait_send()
            rb.wait_send()
            if h < N_DEV - 2:
                grant(ag_credit_f, left)
                grant(ag_credit_b, right)
        cpf_prev.wait()
        cpb_prev.wait()

    return pl.pallas_call(
        body,
        out_shape=jax.ShapeDtypeStruct((M, N), jnp.float32),
        in_specs=[
            pl.BlockSpec(memory_space=pltpu.VMEM),
            pl.BlockSpec(memory_space=pltpu.VMEM),
        ],
        out_specs=pl.BlockSpec(memory_space=pl.ANY),
        scratch_shapes=[
            pltpu.VMEM((CH, H), jnp.float32),
            pltpu.VMEM((CH, H), jnp.float32),
            pltpu.VMEM((2, CH, H), jnp.float32),
            pltpu.VMEM((2, CH, H), jnp.float32),
            pltpu.VMEM((2, CH, H), jnp.float32),
            pltpu.VMEM((2, CH, H), jnp.float32),
            pltpu.SemaphoreType.DMA((2,)),
            pltpu.SemaphoreType.DMA((2,)),
            pltpu.SemaphoreType.DMA((2,)),
            pltpu.SemaphoreType.DMA((2,)),
            pltpu.SemaphoreType.DMA((2,)),
            pltpu.SemaphoreType.DMA((2,)),
            pltpu.SemaphoreType.DMA((2,)),
            pltpu.SemaphoreType.DMA((2,)),
            pltpu.SemaphoreType.REGULAR,
            pltpu.SemaphoreType.REGULAR,
            pltpu.SemaphoreType.REGULAR,
            pltpu.SemaphoreType.REGULAR,
            pltpu.SemaphoreType.DMA((2,)),
            pltpu.SemaphoreType.DMA((2,)),
        ],
        compiler_params=pltpu.CompilerParams(
            collective_id=0,
            vmem_limit_bytes=100 * 1024 * 1024,
        ),
    )(x, w_mat)


# device time: 1608583 ns/iter; 1.8735x vs baseline; 1.0058x over previous
import jax
import jax.numpy as jnp
from jax import lax
from jax.experimental import pallas as pl
from jax.experimental.pallas import tpu as pltpu

N_DEV = 16


def kernel(x, w_mat):
    M, K = x.shape
    _, N = w_mat.shape
    CH = M // N_DEV
    H = N // 2

    def body(x_ref, w_ref, out_ref,
             acc_f, acc_b, rs_f, rs_b, ag_f, ag_b,
             rs_send_f, rs_recv_f, rs_send_b, rs_recv_b,
             ag_send_f, ag_recv_f, ag_send_b, ag_recv_b,
             rs_credit_f, rs_credit_b, ag_credit_f, ag_credit_b,
             store_f, store_b):
        d = lax.axis_index("i")
        left = (d + N_DEV - 1) % N_DEV
        right = (d + 1) % N_DEV

        barrier = pltpu.get_barrier_semaphore()
        for nbr in (left, right):
            pl.semaphore_signal(barrier, inc=1, device_id=(nbr,),
                                device_id_type=pl.DeviceIdType.MESH)
        pl.semaphore_wait(barrier, 2)

        def grant(sem, to):
            pl.semaphore_signal(sem, inc=1, device_id=(to,),
                                device_id_type=pl.DeviceIdType.MESH)

        grant(rs_credit_f, left)
        grant(rs_credit_b, right)
        grant(ag_credit_f, left)
        grant(ag_credit_b, right)

        def pchunk(c, lo):
            return jnp.dot(x_ref[pl.ds(c * CH, CH), :], w_ref[:, lo:lo + H],
                           preferred_element_type=jnp.float32)

        def rcopy(buf, send_sems, recv_sems, send_slot, recv_slot, to):
            return pltpu.make_async_remote_copy(
                src_ref=buf.at[send_slot],
                dst_ref=buf.at[recv_slot],
                send_sem=send_sems.at[send_slot],
                recv_sem=recv_sems.at[recv_slot],
                device_id=(to,),
                device_id_type=pl.DeviceIdType.MESH,
            )

        def rs_desc_f(s):
            return rcopy(rs_f, rs_send_f, rs_recv_f, (s + 1) % 2, s % 2,
                         right)

        def rs_desc_b(s):
            return rcopy(rs_b, rs_send_b, rs_recv_b, (s + 1) % 2, s % 2,
                         left)

        rs_f[1, :, :] = pchunk(d, 0)
        rs_b[1, :, :] = pchunk(d, H)
        pl.semaphore_wait(rs_credit_f, 1)
        rs_desc_f(0).start()
        pl.semaphore_wait(rs_credit_b, 1)
        rs_desc_b(0).start()
        for s in range(N_DEV - 1):
            recv_slot = s % 2
            cf = (d - s - 1) % N_DEV
            cb = (d + s + 1) % N_DEV
            acc_f[:, :] = pchunk(cf, 0)
            acc_b[:, :] = pchunk(cb, H)
            df = rs_desc_f(s)
            db = rs_desc_b(s)
            df.wait_recv()
            df.wait_send()
            if s < N_DEV - 2:
                grant(rs_credit_f, left)
            rs_f[recv_slot, :, :] = rs_f[recv_slot, :, :] + acc_f[:, :]
            if s < N_DEV - 2:
                pl.semaphore_wait(rs_credit_f, 1)
                rs_desc_f(s + 1).start()
            db.wait_recv()
            db.wait_send()
            if s < N_DEV - 2:
                grant(rs_credit_b, right)
            rs_b[recv_slot, :, :] = rs_b[recv_slot, :, :] + acc_b[:, :]
            if s < N_DEV - 2:
                pl.semaphore_wait(rs_credit_b, 1)
                rs_desc_b(s + 1).start()

        own_f = (d + 1) % N_DEV
        own_b = (d + N_DEV - 1) % N_DEV
        vf = rs_f[0, :, :]
        ag_f[0, :, :] = vf * (1.0 / (1.0 + jnp.exp(-vf)))
        vb = rs_b[0, :, :]
        ag_b[0, :, :] = vb * (1.0 / (1.0 + jnp.exp(-vb)))
        cpf_prev = pltpu.make_async_copy(
            ag_f.at[0], out_ref.at[pl.ds(own_f * CH, CH), pl.ds(0, H)],
            store_f.at[1])
        cpb_prev = pltpu.make_async_copy(
            ag_b.at[0], out_ref.at[pl.ds(own_b * CH, CH), pl.ds(H, H)],
            store_b.at[1])
        cpf_prev.start()
        cpb_prev.start()

        for h in range(N_DEV - 1):
            send_slot = h % 2
            recv_slot = (h + 1) % 2
            cf = (d - h) % N_DEV
            cb = (d + h) % N_DEV
            pl.semaphore_wait(ag_credit_f, 1)
            rf = rcopy(ag_f, ag_send_f, ag_recv_f, send_slot, recv_slot, right)
            rf.start()
            pl.semaphore_wait(ag_credit_b, 1)
            rb = rcopy(ag_b, ag_send_b, ag_recv_b, send_slot, recv_slot, left)
            rb.start()
            rf.wait_recv()
            cpf = pltpu.make_async_copy(
                ag_f.at[recv_slot],
                out_ref.at[pl.ds(cf * CH, CH), pl.ds(0, H)],
                store_f.at[h % 2])
            cpf.start()
            rb.wait_recv()
            cpb = pltpu.make_async_copy(
                ag_b.at[recv_slot],
                out_ref.at[pl.ds(cb * CH, CH), pl.ds(H, H)],
                store_b.at[h % 2])
            cpb.start()
            cpf_prev.wait()
            cpb_prev.wait()
            cpf_prev, cpb_prev = cpf, cpb
            rf.wait_send()
            rb.wait_send()
            if h < N_DEV - 2:
                grant(ag_credit_f, left)
                grant(ag_credit_b, right)
        cpf_prev.wait()
        cpb_prev.wait()

    return pl.pallas_call(
        body,
        out_shape=jax.ShapeDtypeStruct((M, N), jnp.float32),
        in_specs=[
            pl.BlockSpec(memory_space=pltpu.VMEM),
            pl.BlockSpec(memory_space=pltpu.VMEM),
        ],
        out_specs=pl.BlockSpec(memory_space=pl.ANY),
        scratch_shapes=[
            pltpu.VMEM((CH, H), jnp.float32),
            pltpu.VMEM((CH, H), jnp.float32),
            pltpu.VMEM((2, CH, H), jnp.float32),
            pltpu.VMEM((2, CH, H), jnp.float32),
            pltpu.VMEM((2, CH, H), jnp.float32),
            pltpu.VMEM((2, CH, H), jnp.float32),
            pltpu.SemaphoreType.DMA((2,)),
            pltpu.SemaphoreType.DMA((2,)),
            pltpu.SemaphoreType.DMA((2,)),
            pltpu.SemaphoreType.DMA((2,)),
            pltpu.SemaphoreType.DMA((2,)),
            pltpu.SemaphoreType.DMA((2,)),
            pltpu.SemaphoreType.DMA((2,)),
            pltpu.SemaphoreType.DMA((2,)),
            pltpu.SemaphoreType.REGULAR,
            pltpu.SemaphoreType.REGULAR,
            pltpu.SemaphoreType.REGULAR,
            pltpu.SemaphoreType.REGULAR,
            pltpu.SemaphoreType.DMA((2,)),
            pltpu.SemaphoreType.DMA((2,)),
        ],
        compiler_params=pltpu.CompilerParams(
            collective_id=0,
            vmem_limit_bytes=100 * 1024 * 1024,
        ),
    )(x, w_mat)
